# baseline (speedup 1.0000x reference)
import functools

import jax
import jax.numpy as jnp
from jax import lax
from jax.experimental import pallas as pl
from jax.experimental.pallas import tpu as pltpu
from jax.experimental.pallas import tpu_sc as plsc

_B = 4
_S = 8192
_D = 1024
_NW = 32
_ROWS_PER_W = _S // _NW
_CHUNK_ROWS = 8
_NCHUNKS = _ROWS_PER_W // _CHUNK_ROWS
_NVREG = _CHUNK_ROWS * _D // 16
_NBUF = 3

_mesh = plsc.VectorSubcoreMesh(core_axis_name="c", subcore_axis_name="s")


@functools.partial(
    pl.kernel,
    out_type=jax.ShapeDtypeStruct((_B * _S, _D), jnp.float32),
    mesh=_mesh,
    scratch_types=[
        pltpu.VMEM((_NBUF, _CHUNK_ROWS, _D), jnp.float32),
        pltpu.VMEM((_NBUF, _B, _CHUNK_ROWS, _D), jnp.float32),
        pltpu.SemaphoreType.DMA,
        pltpu.SemaphoreType.DMA,
        pltpu.SemaphoreType.DMA,
        pltpu.SemaphoreType.DMA,
        pltpu.SemaphoreType.DMA,
        pltpu.SemaphoreType.DMA,
    ],
)
def _pos_add(x_hbm, pos_hbm, out_hbm, pos_buf, x_buf,
             sin0, sin1, sin2, sout0, sout1, sout2):
    cid = lax.axis_index("c")
    sid = lax.axis_index("s")
    wid = sid * 2 + cid
    base = wid * _ROWS_PER_W
    sin = (sin0, sin1, sin2)
    sout = (sout0, sout1, sout2)

    def load_descs(c, p):
        row = base + c * _CHUNK_ROWS
        descs = [pltpu.make_async_copy(
            pos_hbm.at[pl.ds(row, _CHUNK_ROWS)], pos_buf.at[p], sin[p])]
        for b in range(_B):
            descs.append(pltpu.make_async_copy(
                x_hbm.at[pl.ds(b * _S + row, _CHUNK_ROWS)],
                x_buf.at[p, b], sin[p]))
        return descs

    def store_descs(c, p):
        row = base + c * _CHUNK_ROWS
        return [pltpu.make_async_copy(
            x_buf.at[p, b], out_hbm.at[pl.ds(b * _S + row, _CHUNK_ROWS)],
            sout[p]) for b in range(_B)]

    def compute(p):
        @plsc.parallel_loop(0, _NVREG, unroll=8)
        def _(j):
            r = j // (_D // 16)
            sl = pl.ds((j % (_D // 16)) * 16, 16)
            pv = pos_buf[p, r, sl]
            for b in range(_B):
                x_buf[p, b, r, sl] = x_buf[p, b, r, sl] + pv

    for d in load_descs(0, 0):
        d.start()
    for d in load_descs(1, 1):
        d.start()
    for c in range(_NCHUNKS):
        p = c % _NBUF
        for d in load_descs(c, p):
            d.wait()
        compute(p)
        for d in store_descs(c, p):
            d.start()
        if c + 2 < _NCHUNKS:
            q = (c + 2) % _NBUF
            if c >= 1:
                for d in store_descs(c - 1, q):
                    d.wait()
            for d in load_descs(c + 2, q):
                d.start()
    for c in range(_NCHUNKS - 3, _NCHUNKS):
        for d in store_descs(c, c % _NBUF):
            d.wait()


@jax.jit
def kernel(x, pos_table):
    out = _pos_add(x.reshape(_B * _S, _D), pos_table)
    return out.reshape(x.shape)


# speedup vs baseline: 1.0642x; 1.0642x over previous
import functools

import jax
import jax.numpy as jnp
from jax import lax
from jax.experimental import pallas as pl
from jax.experimental.pallas import tpu as pltpu
from jax.experimental.pallas import tpu_sc as plsc

_B = 4
_S = 8192
_D = 1024
_NW = 32
_ROWS_PER_W = _S // _NW
_CHUNK_ROWS = 8
_NCHUNKS = _ROWS_PER_W // _CHUNK_ROWS
_NVREG = _CHUNK_ROWS * _D // 16
_NBUF = 3
_NSTEADY = (_NCHUNKS - 2) // _NBUF

_mesh = plsc.VectorSubcoreMesh(core_axis_name="c", subcore_axis_name="s")


@functools.partial(
    pl.kernel,
    out_type=jax.ShapeDtypeStruct((_B * _S, _D), jnp.float32),
    mesh=_mesh,
    scratch_types=[
        pltpu.VMEM((_NBUF, _CHUNK_ROWS, _D), jnp.float32),
        pltpu.VMEM((_NBUF, _B, _CHUNK_ROWS, _D), jnp.float32),
        pltpu.SemaphoreType.DMA,
        pltpu.SemaphoreType.DMA,
        pltpu.SemaphoreType.DMA,
        pltpu.SemaphoreType.DMA,
        pltpu.SemaphoreType.DMA,
        pltpu.SemaphoreType.DMA,
    ],
)
def _pos_add(x_hbm, pos_hbm, out_hbm, pos_buf, x_buf,
             sin0, sin1, sin2, sout0, sout1, sout2):
    cid = lax.axis_index("c")
    sid = lax.axis_index("s")
    wid = sid * 2 + cid
    base = wid * _ROWS_PER_W
    sin = (sin0, sin1, sin2)
    sout = (sout0, sout1, sout2)

    def load_descs(c, p):
        row = base + c * _CHUNK_ROWS
        descs = [pltpu.make_async_copy(
            pos_hbm.at[pl.ds(row, _CHUNK_ROWS)], pos_buf.at[p], sin[p])]
        for b in range(_B):
            descs.append(pltpu.make_async_copy(
                x_hbm.at[pl.ds(b * _S + row, _CHUNK_ROWS)],
                x_buf.at[p, b], sin[p]))
        return descs

    def store_descs(c, p):
        row = base + c * _CHUNK_ROWS
        return [pltpu.make_async_copy(
            x_buf.at[p, b], out_hbm.at[pl.ds(b * _S + row, _CHUNK_ROWS)],
            sout[p]) for b in range(_B)]

    def compute(p):
        @plsc.parallel_loop(0, _NVREG, unroll=8)
        def _(j):
            r = j // (_D // 16)
            sl = pl.ds((j % (_D // 16)) * 16, 16)
            pv = pos_buf[p, r, sl]
            for b in range(_B):
                x_buf[p, b, r, sl] = x_buf[p, b, r, sl] + pv

    for d in load_descs(0, 0):
        d.start()
    for d in load_descs(1, 1):
        d.start()
    for d in load_descs(0, 0):
        d.wait()
    compute(0)
    for d in store_descs(0, 0):
        d.start()
    for d in load_descs(2, 2):
        d.start()

    def stage(c, p):
        q = (p + 2) % _NBUF
        for d in load_descs(c, p):
            d.wait()
        compute(p)
        for d in store_descs(c, p):
            d.start()
        for d in store_descs(c - 1, q):
            d.wait()
        for d in load_descs(c + 2, q):
            d.start()

    def group(k, carry):
        for pp in range(_NBUF):
            stage(_NBUF * k + 1 + pp, (1 + pp) % _NBUF)
        return carry

    lax.fori_loop(0, _NSTEADY - 1, group, 0)

    for pp in range(_NBUF - 1):
        c = _NBUF * (_NSTEADY - 1) + 1 + pp
        stage(c, c % _NBUF)
    for c in range(_NCHUNKS - 2, _NCHUNKS):
        p = c % _NBUF
        for d in load_descs(c, p):
            d.wait()
        compute(p)
        for d in store_descs(c, p):
            d.start()
    for c in range(_NCHUNKS - 3, _NCHUNKS):
        for d in store_descs(c, c % _NBUF):
            d.wait()


@jax.jit
def kernel(x, pos_table):
    out = _pos_add(x.reshape(_B * _S, _D), pos_table)
    return out.reshape(x.shape)


# speedup vs baseline: 1.0674x; 1.0030x over previous
import functools

import jax
import jax.numpy as jnp
from jax import lax
from jax.experimental import pallas as pl
from jax.experimental.pallas import tpu as pltpu
from jax.experimental.pallas import tpu_sc as plsc

_B = 4
_S = 8192
_D = 1024
_NW = 32
_ROWS_PER_W = _S // _NW
_CHUNK_ROWS = 8
_NCHUNKS = _ROWS_PER_W // _CHUNK_ROWS
_NVREG = _CHUNK_ROWS * _D // 16
_NBUF = 3
_NSTEADY = (_NCHUNKS - 2) // _NBUF

_mesh = plsc.VectorSubcoreMesh(core_axis_name="c", subcore_axis_name="s")


@functools.partial(
    pl.kernel,
    out_type=jax.ShapeDtypeStruct((_B * _S, _D), jnp.float32),
    mesh=_mesh,
    scratch_types=[
        pltpu.VMEM((_NBUF, _CHUNK_ROWS, _D), jnp.float32),
        pltpu.VMEM((_NBUF, _B, _CHUNK_ROWS, _D), jnp.float32),
        pltpu.SemaphoreType.DMA,
        pltpu.SemaphoreType.DMA,
        pltpu.SemaphoreType.DMA,
        pltpu.SemaphoreType.DMA,
        pltpu.SemaphoreType.DMA,
        pltpu.SemaphoreType.DMA,
    ],
)
def _pos_add(x_hbm, pos_hbm, out_hbm, pos_buf, x_buf,
             sin0, sin1, sin2, sout0, sout1, sout2):
    cid = lax.axis_index("c")
    sid = lax.axis_index("s")
    wid = sid * 2 + cid
    base = wid * _ROWS_PER_W
    sin = (sin0, sin1, sin2)
    sout = (sout0, sout1, sout2)

    def load_descs(c, p):
        row = base + c * _CHUNK_ROWS
        descs = [pltpu.make_async_copy(
            pos_hbm.at[pl.ds(row, _CHUNK_ROWS)], pos_buf.at[p], sin[p])]
        for b in range(_B):
            descs.append(pltpu.make_async_copy(
                x_hbm.at[pl.ds(b * _S + row, _CHUNK_ROWS)],
                x_buf.at[p, b], sin[p]))
        return descs

    def store_descs(c, p):
        row = base + c * _CHUNK_ROWS
        return [pltpu.make_async_copy(
            x_buf.at[p, b], out_hbm.at[pl.ds(b * _S + row, _CHUNK_ROWS)],
            sout[p]) for b in range(_B)]

    def compute(p):
        @plsc.parallel_loop(0, _NVREG, unroll=4)
        def _(j):
            r = j // (_D // 16)
            sl = pl.ds((j % (_D // 16)) * 16, 16)
            pv = pos_buf[p, r, sl]
            for b in range(_B):
                x_buf[p, b, r, sl] = x_buf[p, b, r, sl] + pv

    for d in load_descs(0, 0):
        d.start()
    for d in load_descs(1, 1):
        d.start()
    for d in load_descs(0, 0):
        d.wait()
    compute(0)
    for d in store_descs(0, 0):
        d.start()
    for d in load_descs(2, 2):
        d.start()

    def stage(c, p):
        q = (p + 2) % _NBUF
        for d in load_descs(c, p):
            d.wait()
        compute(p)
        for d in store_descs(c, p):
            d.start()
        for d in store_descs(c - 1, q):
            d.wait()
        for d in load_descs(c + 2, q):
            d.start()

    def group(k, carry):
        for pp in range(_NBUF):
            stage(_NBUF * k + 1 + pp, (1 + pp) % _NBUF)
        return carry

    lax.fori_loop(0, _NSTEADY - 1, group, 0)

    for pp in range(_NBUF - 1):
        c = _NBUF * (_NSTEADY - 1) + 1 + pp
        stage(c, c % _NBUF)
    for c in range(_NCHUNKS - 2, _NCHUNKS):
        p = c % _NBUF
        for d in load_descs(c, p):
            d.wait()
        compute(p)
        for d in store_descs(c, p):
            d.start()
    for c in range(_NCHUNKS - 3, _NCHUNKS):
        for d in store_descs(c, c % _NBUF):
            d.wait()


@jax.jit
def kernel(x, pos_table):
    out = _pos_add(x.reshape(_B * _S, _D), pos_table)
    return out.reshape(x.shape)


# speedup vs baseline: 1.0688x; 1.0013x over previous
import functools

import jax
import jax.numpy as jnp
from jax import lax
from jax.experimental import pallas as pl
from jax.experimental.pallas import tpu as pltpu
from jax.experimental.pallas import tpu_sc as plsc

_B = 4
_S = 8192
_D = 1024
_NW = 32
_ROWS_PER_W = _S // _NW
_CHUNK_ROWS = 8
_NCHUNKS = _ROWS_PER_W // _CHUNK_ROWS
_NVREG = _CHUNK_ROWS * _D // 16
_NBUF = 3
_NSTEADY = (_NCHUNKS - 2) // _NBUF

_mesh = plsc.VectorSubcoreMesh(core_axis_name="c", subcore_axis_name="s")


@functools.partial(
    pl.kernel,
    out_type=jax.ShapeDtypeStruct((_B, _S, _D), jnp.float32),
    mesh=_mesh,
    scratch_types=[
        pltpu.VMEM((_NBUF, _CHUNK_ROWS, _D), jnp.float32),
        pltpu.VMEM((_NBUF, _B, _CHUNK_ROWS, _D), jnp.float32),
        pltpu.SemaphoreType.DMA,
        pltpu.SemaphoreType.DMA,
        pltpu.SemaphoreType.DMA,
        pltpu.SemaphoreType.DMA,
        pltpu.SemaphoreType.DMA,
        pltpu.SemaphoreType.DMA,
    ],
)
def _pos_add(x_hbm, pos_hbm, out_hbm, pos_buf, x_buf,
             sin0, sin1, sin2, sout0, sout1, sout2):
    cid = lax.axis_index("c")
    sid = lax.axis_index("s")
    wid = sid * 2 + cid
    base = wid * _ROWS_PER_W
    sin = (sin0, sin1, sin2)
    sout = (sout0, sout1, sout2)

    def load_descs(c, p):
        row = base + c * _CHUNK_ROWS
        return [
            pltpu.make_async_copy(
                pos_hbm.at[pl.ds(row, _CHUNK_ROWS)], pos_buf.at[p], sin[p]),
            pltpu.make_async_copy(
                x_hbm.at[:, pl.ds(row, _CHUNK_ROWS), :], x_buf.at[p],
                sin[p]),
        ]

    def store_descs(c, p):
        row = base + c * _CHUNK_ROWS
        return [pltpu.make_async_copy(
            x_buf.at[p], out_hbm.at[:, pl.ds(row, _CHUNK_ROWS), :],
            sout[p])]

    def compute(p):
        @plsc.parallel_loop(0, _NVREG, unroll=4)
        def _(j):
            r = j // (_D // 16)
            sl = pl.ds((j % (_D // 16)) * 16, 16)
            pv = pos_buf[p, r, sl]
            for b in range(_B):
                x_buf[p, b, r, sl] = x_buf[p, b, r, sl] + pv

    for d in load_descs(0, 0):
        d.start()
    for d in load_descs(1, 1):
        d.start()
    for d in load_descs(0, 0):
        d.wait()
    compute(0)
    for d in store_descs(0, 0):
        d.start()
    for d in load_descs(2, 2):
        d.start()

    def stage(c, p):
        q = (p + 2) % _NBUF
        for d in load_descs(c, p):
            d.wait()
        compute(p)
        for d in store_descs(c, p):
            d.start()
        for d in store_descs(c - 1, q):
            d.wait()
        for d in load_descs(c + 2, q):
            d.start()

    def group(k, carry):
        for pp in range(_NBUF):
            stage(_NBUF * k + 1 + pp, (1 + pp) % _NBUF)
        return carry

    lax.fori_loop(0, _NSTEADY - 1, group, 0)

    for pp in range(_NBUF - 1):
        c = _NBUF * (_NSTEADY - 1) + 1 + pp
        stage(c, c % _NBUF)
    for c in range(_NCHUNKS - 2, _NCHUNKS):
        p = c % _NBUF
        for d in load_descs(c, p):
            d.wait()
        compute(p)
        for d in store_descs(c, p):
            d.start()
    for c in range(_NCHUNKS - 3, _NCHUNKS):
        for d in store_descs(c, c % _NBUF):
            d.wait()


@jax.jit
def kernel(x, pos_table):
    return _pos_add(x, pos_table)
